# initial kernel scaffold (unmeasured)
import jax
import jax.numpy as jnp
from jax import lax
from jax.experimental import pallas as pl
from jax.experimental.pallas import tpu as pltpu

P = 32


def kernel(x, w_mat):
    m, k = x.shape
    _, n = w_mat.shape
    nb = n // P

    def body(x_ref, w_ref, out_ref, xb, wbuf, send_buf, recv_buf,
             wcopy_sems, send_sems, recv_sems):
        me = lax.axis_index("i")

        bar = pltpu.get_barrier_semaphore()
        for p in range(1, P):
            pl.semaphore_signal(
                bar, inc=1,
                device_id=((me + p) % P,),
                device_id_type=pl.DeviceIdType.MESH,
            )
        pl.semaphore_wait(bar, P - 1)

        xb[...] = x_ref[...].astype(jnp.bfloat16)

        def w_fetch(s, slot):
            t = (me + s) % P
            return pltpu.make_async_copy(
                w_ref.at[:, pl.ds(t * nb, nb)], wbuf.at[slot],
                wcopy_sems.at[slot],
            )

        w_fetch(0, 0).start()

        sends = []
        for s in range(P):
            t = (me + s) % P
            if s + 1 < P:
                w_fetch(s + 1, (s + 1) % 2).start()
            w_fetch(s, s % 2).wait()
            wb = wbuf[s % 2].astype(jnp.bfloat16)
            y = lax.dot_general(
                xb[...], wb, (((1,), (0,)), ((), ())),
                preferred_element_type=jnp.float32,
            )
            y = y * jax.nn.sigmoid(y)
            if s == 0:
                out_ref[pl.ds(me * m, m), :] = y
            else:
                send_buf[s] = y.astype(jnp.bfloat16)
                rdma = pltpu.make_async_remote_copy(
                    src_ref=send_buf.at[s],
                    dst_ref=recv_buf.at[s],
                    send_sem=send_sems.at[s],
                    recv_sem=recv_sems.at[s],
                    device_id=(t,),
                    device_id_type=pl.DeviceIdType.MESH,
                )
                rdma.start()
                sends.append(rdma)

        for s in range(1, P):
            src = (me - s) % P
            sends[s - 1].wait_recv()
            out_ref[pl.ds(src * m, m), :] = recv_buf[s].astype(jnp.float32)
        for rdma in sends:
            rdma.wait_send()

    return pl.pallas_call(
        body,
        out_shape=jax.ShapeDtypeStruct((P * m, nb), jnp.float32),
        in_specs=[
            pl.BlockSpec(memory_space=pltpu.VMEM),
            pl.BlockSpec(memory_space=pltpu.ANY),
        ],
        out_specs=pl.BlockSpec(memory_space=pltpu.VMEM),
        scratch_shapes=[
            pltpu.VMEM((m, k), jnp.bfloat16),
            pltpu.VMEM((2, k, nb), jnp.float32),
            pltpu.VMEM((P, m, nb), jnp.bfloat16),
            pltpu.VMEM((P, m, nb), jnp.bfloat16),
            pltpu.SemaphoreType.DMA((2,)),
            pltpu.SemaphoreType.DMA((P,)),
            pltpu.SemaphoreType.DMA((P,)),
        ],
        compiler_params=pltpu.CompilerParams(collective_id=0),
    )(x, w_mat)


# baseline (device time: 77731 ns/iter reference)
import jax
import jax.numpy as jnp
from jax import lax
from jax.experimental import pallas as pl
from jax.experimental.pallas import tpu as pltpu

P = 32


def kernel(x, w_mat):
    m, k = x.shape
    _, n = w_mat.shape
    nb = n // P

    def body(x_ref, w_ref, out_ref, xb, wbuf, send_buf, recv_buf,
             wcopy_sems, send_sems, recv_sems):
        me = lax.axis_index("i")

        bar = pltpu.get_barrier_semaphore()
        for p in range(1, P):
            pl.semaphore_signal(
                bar, inc=1,
                device_id=((me + p) % P,),
                device_id_type=pl.DeviceIdType.MESH,
            )
        pl.semaphore_wait(bar, P - 1)

        xb[...] = x_ref[...].astype(jnp.bfloat16)

        def w_fetch(s, slot):
            t = (me + s) % P
            return pltpu.make_async_copy(
                w_ref.at[:, pl.ds(t * nb, nb)], wbuf.at[slot],
                wcopy_sems.at[slot],
            )

        def a2a_rdma(s):
            t = (me + s) % P
            return pltpu.make_async_remote_copy(
                src_ref=send_buf.at[s],
                dst_ref=recv_buf.at[s],
                send_sem=send_sems.at[s],
                recv_sem=recv_sems.at[s],
                device_id=(t,),
                device_id_type=pl.DeviceIdType.MESH,
            )

        w_fetch(0, 0).start()

        def send_step(s, carry):
            slot = s % 2

            @pl.when(s < P - 1)
            def _():
                w_fetch(s + 1, (s + 1) % 2).start()

            w_fetch(s, slot).wait()
            wb = wbuf[slot].astype(jnp.bfloat16)
            y = lax.dot_general(
                xb[...], wb, (((1,), (0,)), ((), ())),
                preferred_element_type=jnp.float32,
            )
            y = y * jax.nn.sigmoid(y)

            @pl.when(s == 0)
            def _():
                out_ref[pl.ds(me * m, m), :] = y

            @pl.when(s > 0)
            def _():
                send_buf[s] = y.astype(jnp.bfloat16)
                a2a_rdma(s).start()

            return carry

        lax.fori_loop(0, P, send_step, 0)

        def recv_step(s, carry):
            src = (me - s) % P
            rdma = a2a_rdma(s)
            rdma.wait_recv()
            out_ref[pl.ds(src * m, m), :] = recv_buf[s].astype(jnp.float32)
            rdma.wait_send()
            return carry

        lax.fori_loop(1, P, recv_step, 0)

    return pl.pallas_call(
        body,
        out_shape=jax.ShapeDtypeStruct((P * m, nb), jnp.float32),
        in_specs=[
            pl.BlockSpec(memory_space=pltpu.VMEM),
            pl.BlockSpec(memory_space=pl.ANY),
        ],
        out_specs=pl.BlockSpec(memory_space=pltpu.VMEM),
        scratch_shapes=[
            pltpu.VMEM((m, k), jnp.bfloat16),
            pltpu.VMEM((2, k, nb), jnp.float32),
            pltpu.VMEM((P, m, nb), jnp.bfloat16),
            pltpu.VMEM((P, m, nb), jnp.bfloat16),
            pltpu.SemaphoreType.DMA((2,)),
            pltpu.SemaphoreType.DMA((P,)),
            pltpu.SemaphoreType.DMA((P,)),
        ],
        compiler_params=pltpu.CompilerParams(collective_id=0),
    )(x, w_mat)


# device time: 62600 ns/iter; 1.2417x vs baseline; 1.2417x over previous
import jax
import jax.numpy as jnp
from jax import lax
from jax.experimental import pallas as pl
from jax.experimental.pallas import tpu as pltpu

P = 32
G = 4
KC = 4


def kernel(x, w_mat):
    m, k = x.shape
    _, n = w_mat.shape
    nb = n // P
    cb = n // G
    kb = k // KC
    bpg = cb // nb

    def body(x_ref, w_ref, out_ref, xb, wbuf, acc, send_buf, recv_buf,
             wcopy_sems, send_sems, recv_sems):
        me = lax.axis_index("i")

        bar = pltpu.get_barrier_semaphore()
        for p in range(1, P):
            pl.semaphore_signal(
                bar, inc=1,
                device_id=((me + p) % P,),
                device_id_type=pl.DeviceIdType.MESH,
            )
        pl.semaphore_wait(bar, P - 1)

        xb[...] = x_ref[...].astype(jnp.bfloat16)

        def colgrp(g):
            return (me // (P // G) + g) % G

        def w_fetch(f, slot):
            g, kc = f // KC, f % KC
            cg = colgrp(g)
            return pltpu.make_async_copy(
                w_ref.at[pl.ds(kc * kb, kb), pl.ds(cg * cb, cb)],
                wbuf.at[slot], wcopy_sems.at[slot],
            )

        w_fetch(0, 0).start()

        def step(f, carry):
            g, kc = f // KC, f % KC
            slot = f % 2

            @pl.when(f < G * KC - 1)
            def _():
                w_fetch(f + 1, (f + 1) % 2).start()

            w_fetch(f, slot).wait()
            wb = wbuf[slot].astype(jnp.bfloat16)
            partial = lax.dot_general(
                xb[:, pl.ds(kc * kb, kb)], wb, (((1,), (0,)), ((), ())),
                preferred_element_type=jnp.float32,
            )

            @pl.when(kc == 0)
            def _():
                acc[...] = partial

            @pl.when(kc != 0)
            def _():
                acc[...] = acc[...] + partial

            @pl.when(kc == KC - 1)
            def _():
                y = acc[...]
                y = y * jax.nn.sigmoid(y)
                cg = colgrp(g)
                for b in range(bpg):
                    t = cg * bpg + b
                    blk = y[:, b * nb:(b + 1) * nb]

                    @pl.when(t == me)
                    def _():
                        out_ref[pl.ds(me * m, m), :] = blk

                    @pl.when(t != me)
                    def _():
                        send_buf[t] = blk.astype(jnp.bfloat16)
                        pltpu.make_async_remote_copy(
                            src_ref=send_buf.at[t],
                            dst_ref=recv_buf.at[me],
                            send_sem=send_sems.at[t],
                            recv_sem=recv_sems.at[me],
                            device_id=(t,),
                            device_id_type=pl.DeviceIdType.MESH,
                        ).start()

            return carry

        lax.fori_loop(0, G * KC, step, 0)

        def recv_step(s, carry):
            src = (me - s) % P
            t = (me + s) % P
            rdma = pltpu.make_async_remote_copy(
                src_ref=send_buf.at[t],
                dst_ref=recv_buf.at[src],
                send_sem=send_sems.at[t],
                recv_sem=recv_sems.at[src],
                device_id=(t,),
                device_id_type=pl.DeviceIdType.MESH,
            )
            rdma.wait_recv()
            out_ref[pl.ds(src * m, m), :] = recv_buf[src].astype(jnp.float32)
            rdma.wait_send()
            return carry

        lax.fori_loop(1, P, recv_step, 0)

    return pl.pallas_call(
        body,
        out_shape=jax.ShapeDtypeStruct((P * m, nb), jnp.float32),
        in_specs=[
            pl.BlockSpec(memory_space=pltpu.VMEM),
            pl.BlockSpec(memory_space=pl.ANY),
        ],
        out_specs=pl.BlockSpec(memory_space=pltpu.VMEM),
        scratch_shapes=[
            pltpu.VMEM((m, k), jnp.bfloat16),
            pltpu.VMEM((2, kb, cb), jnp.float32),
            pltpu.VMEM((m, cb), jnp.float32),
            pltpu.VMEM((P, m, nb), jnp.bfloat16),
            pltpu.VMEM((P, m, nb), jnp.bfloat16),
            pltpu.SemaphoreType.DMA((2,)),
            pltpu.SemaphoreType.DMA((P,)),
            pltpu.SemaphoreType.DMA((P,)),
        ],
        compiler_params=pltpu.CompilerParams(collective_id=0),
    )(x, w_mat)
